# baseline (speedup 1.0000x reference)
import jax
import jax.numpy as jnp
from jax.experimental import pallas as pl

N_SKILLS = 64
BATCH = 16384


def _onehot_kernel(ids_ref, out_ref):
    ids = ids_ref[:]
    r = ids.shape[0]
    iota_j = jax.lax.broadcasted_iota(jnp.int32, (N_SKILLS, 128), 0)
    for k in range(r):
        row = jnp.broadcast_to(ids[k : k + 1, :], (N_SKILLS, 128))
        out_ref[:, k * 128 : (k + 1) * 128] = (row == iota_j).astype(jnp.float32)


def kernel(task_ids):
    ids2 = task_ids.reshape(128, 128).astype(jnp.int32)
    rows_per_block = 64
    out = pl.pallas_call(
        _onehot_kernel,
        grid=(128 // rows_per_block,),
        in_specs=[pl.BlockSpec((rows_per_block, 128), lambda i: (i, 0))],
        out_specs=pl.BlockSpec((N_SKILLS, rows_per_block * 128), lambda i: (0, i)),
        out_shape=jax.ShapeDtypeStruct((N_SKILLS, BATCH), jnp.float32),
    )(ids2)
    return jnp.transpose(out, (1, 0))[:, None, :]
